# Initial kernel scaffold; baseline (speedup 1.0000x reference)
import jax
import jax.numpy as jnp
from jax.experimental import pallas as pl


def kernel(x, edge_index, neg_edge_index, W_self1, W_neigh1, b1, W_self2, W_neigh2, b2):
    raise NotImplementedError("write your pallas kernel here")


# speedup vs baseline: 2.9807x; 2.9807x over previous
import functools

import jax
import jax.numpy as jnp
from jax import lax
from jax.experimental import pallas as pl
from jax.experimental.pallas import tpu as pltpu
from jax.experimental.pallas import tpu_sc as plsc

_DEBUG_JNP_SCORE = False

N = 10000
E = 320000
D = 128

NC = 2
NS = 16
NW = NC * NS

EPW = E // NW
KE = 80
ROWS_PT = 624
ROWS_TAIL = N - NS * ROWS_PT

E2 = 2 * E
SPW = E2 // NW
KS = 80

_MESH = plsc.VectorSubcoreMesh(
    core_axis_name="c", subcore_axis_name="s", num_cores=NC, num_subcores=NS)
_SC_PARAMS = pltpu.CompilerParams(needs_layout_passes=False)


def _agg_body(h, src, dst, zrows, agg_out, sidx, didx, rows, acc, sem):
    cid = lax.axis_index("c")
    sid = lax.axis_index("s")
    wid = cid * NS + sid

    rbase = sid * ROWS_PT
    tbase = NS * ROWS_PT
    pltpu.sync_copy(zrows.at[pl.ds(rbase, ROWS_PT)], acc.at[pl.ds(rbase, ROWS_PT)])

    @pl.when(sid == 0)
    def _():
        pltpu.sync_copy(zrows.at[pl.ds(tbase, ROWS_TAIL)],
                        acc.at[pl.ds(tbase, ROWS_TAIL)])

    plsc.subcore_barrier()

    ebase = wid * EPW

    def blk(j, carry):
        b = ebase + j * KE
        pltpu.sync_copy(src.at[pl.ds(b, KE)], sidx)
        pltpu.sync_copy(dst.at[pl.ds(b, KE)], didx)
        pltpu.async_copy(h.at[sidx], rows, sem).wait()
        pltpu.sync_copy(rows, acc.at[didx], add=True)
        return carry

    lax.fori_loop(0, EPW // KE, blk, 0)
    plsc.subcore_barrier()

    pltpu.sync_copy(acc.at[pl.ds(rbase, ROWS_PT)],
                    agg_out.at[cid].at[pl.ds(rbase, ROWS_PT)])

    @pl.when(sid == 0)
    def _():
        pltpu.sync_copy(acc.at[pl.ds(tbase, ROWS_TAIL)],
                        agg_out.at[cid].at[pl.ds(tbase, ROWS_TAIL)])


_agg = pl.kernel(
    _agg_body,
    out_type=jax.ShapeDtypeStruct((NC, N, D), jnp.float32),
    mesh=_MESH,
    scratch_types=[
        pltpu.VMEM((KE,), jnp.int32),
        pltpu.VMEM((KE,), jnp.int32),
        pltpu.VMEM((KE, D), jnp.float32),
        pltpu.VMEM_SHARED((N, D), jnp.float32),
        pltpu.SemaphoreType.DMA,
    ],
    compiler_params=_SC_PARAMS,
)


def _deg_body(dst, zrows, ones_h, deg_out, didx, onesv, dacc, sem):
    cid = lax.axis_index("c")
    sid = lax.axis_index("s")
    wid = cid * NS + sid

    rbase = sid * ROWS_PT
    tbase = NS * ROWS_PT
    pltpu.sync_copy(zrows.at[pl.ds(rbase, ROWS_PT)], dacc.at[pl.ds(rbase, ROWS_PT)])

    @pl.when(sid == 0)
    def _():
        pltpu.sync_copy(zrows.at[pl.ds(tbase, ROWS_TAIL)],
                        dacc.at[pl.ds(tbase, ROWS_TAIL)])

    pltpu.sync_copy(ones_h, onesv)
    plsc.subcore_barrier()

    ebase = wid * EPW

    def blk(j, carry):
        b = ebase + j * KE
        pltpu.sync_copy(dst.at[pl.ds(b, KE)], didx)
        pltpu.sync_copy(onesv, dacc.at[didx], add=True)
        return carry

    lax.fori_loop(0, EPW // KE, blk, 0)
    plsc.subcore_barrier()

    pltpu.sync_copy(dacc.at[pl.ds(rbase, ROWS_PT)],
                    deg_out.at[cid].at[pl.ds(rbase, ROWS_PT)])

    @pl.when(sid == 0)
    def _():
        pltpu.sync_copy(dacc.at[pl.ds(tbase, ROWS_TAIL)],
                        deg_out.at[cid].at[pl.ds(tbase, ROWS_TAIL)])


_deg = pl.kernel(
    _deg_body,
    out_type=jax.ShapeDtypeStruct((NC, N, D), jnp.float32),
    mesh=_MESH,
    scratch_types=[
        pltpu.VMEM((KE,), jnp.int32),
        pltpu.VMEM((KE, D), jnp.float32),
        pltpu.VMEM_SHARED((N, D), jnp.float32),
        pltpu.SemaphoreType.DMA,
    ],
    compiler_params=_SC_PARAMS,
)


def _score_body(h, srcall, dstall, out, sidx, didx, srows, drows, sv, sem1, sem2):
    cid = lax.axis_index("c")
    sid = lax.axis_index("s")
    wid = cid * NS + sid
    ebase = wid * SPW

    def blk(j, carry):
        b = ebase + j * KS
        pltpu.sync_copy(srcall.at[pl.ds(b, KS)], sidx)
        pltpu.sync_copy(dstall.at[pl.ds(b, KS)], didx)
        c1 = pltpu.async_copy(h.at[sidx], srows, sem1)
        c2 = pltpu.async_copy(h.at[didx], drows, sem2)
        c1.wait()
        c2.wait()
        lane = lax.iota(jnp.int32, 16)
        for g in range(KS // 16):
            score = jnp.zeros((16,), jnp.float32)
            for e in range(16):
                r = g * 16 + e
                p = jnp.zeros((16,), jnp.float32)
                for v in range(D // 16):
                    p = p + (srows[r, pl.ds(v * 16, 16)]
                             * drows[r, pl.ds(v * 16, 16)])
                s = jnp.sum(p)
                score = jnp.where(lane == e, s, score)
            sv[pl.ds(g * 16, 16)] = score
        pltpu.sync_copy(sv, out.at[pl.ds(b, KS)])
        return carry

    lax.fori_loop(0, SPW // KS, blk, 0)


_score = pl.kernel(
    _score_body,
    out_type=jax.ShapeDtypeStruct((E2,), jnp.float32),
    mesh=_MESH,
    scratch_types=[
        pltpu.VMEM((KS,), jnp.int32),
        pltpu.VMEM((KS,), jnp.int32),
        pltpu.VMEM((KS, D), jnp.float32),
        pltpu.VMEM((KS, D), jnp.float32),
        pltpu.VMEM((KS,), jnp.float32),
        pltpu.SemaphoreType.DMA,
        pltpu.SemaphoreType.DMA,
    ],
    compiler_params=_SC_PARAMS,
)


def _combine_body(relu, x, p0, p1, d0, d1, ws, wn, b, out):
    deg = jnp.maximum(d0[:, :1] + d1[:, :1], 1.0)
    hn = (p0[...] + p1[...]) / deg
    r = (jnp.dot(x[...], ws[...], preferred_element_type=jnp.float32,
                 precision=lax.Precision.HIGHEST)
         + jnp.dot(hn, wn[...], preferred_element_type=jnp.float32,
                   precision=lax.Precision.HIGHEST)
         + b[...])
    if relu:
        r = jnp.maximum(r, 0.0)
    out[...] = r


_RB = 1000


def _combine(x, p0, p1, d0, d1, ws, wn, b, relu):
    return pl.pallas_call(
        functools.partial(_combine_body, relu),
        grid=(N // _RB,),
        in_specs=[
            pl.BlockSpec((_RB, D), lambda i: (i, 0)),
            pl.BlockSpec((_RB, D), lambda i: (i, 0)),
            pl.BlockSpec((_RB, D), lambda i: (i, 0)),
            pl.BlockSpec((_RB, D), lambda i: (i, 0)),
            pl.BlockSpec((_RB, D), lambda i: (i, 0)),
            pl.BlockSpec((D, D), lambda i: (0, 0)),
            pl.BlockSpec((D, D), lambda i: (0, 0)),
            pl.BlockSpec((1, D), lambda i: (0, 0)),
        ],
        out_specs=pl.BlockSpec((_RB, D), lambda i: (i, 0)),
        out_shape=jax.ShapeDtypeStruct((N, D), jnp.float32),
    )(x, p0, p1, d0, d1, ws, wn, b)


def kernel(x, edge_index, neg_edge_index, W_self1, W_neigh1, b1,
           W_self2, W_neigh2, b2):
    src = edge_index[0]
    dst = edge_index[1]
    zrows = jnp.zeros((N, D), jnp.float32)
    ones_h = jnp.ones((KE, D), jnp.float32)

    agg1 = _agg(x, src, dst, zrows)
    deg = _deg(dst, zrows, ones_h)
    h1 = _combine(x, agg1[0], agg1[1], deg[0], deg[1],
                  W_self1, W_neigh1, b1.reshape(1, D), relu=True)
    agg2 = _agg(h1, src, dst, zrows)
    h2 = _combine(h1, agg2[0], agg2[1], deg[0], deg[1],
                  W_self2, W_neigh2, b2.reshape(1, D), relu=False)

    src_all = jnp.concatenate([src, neg_edge_index[0]])
    dst_all = jnp.concatenate([dst, neg_edge_index[1]])
    if _DEBUG_JNP_SCORE:
        hu = jnp.take(h2, src_all, axis=0)
        hv = jnp.take(h2, dst_all, axis=0)
        scores = jnp.sum(hu * hv, axis=-1)
    else:
        scores = _score(h2, src_all, dst_all)
    return (scores[:E, None], scores[E:, None])


# speedup vs baseline: 5.9429x; 1.9938x over previous
import functools

import jax
import jax.numpy as jnp
from jax import lax
from jax.experimental import pallas as pl
from jax.experimental.pallas import tpu as pltpu
from jax.experimental.pallas import tpu_sc as plsc

N = 10000
E = 320000
D = 128

NC = 2
NS = 16
NW = NC * NS

EPW = E // NW
KE = 80
NBE = EPW // KE
ROWS_PT = 624
ROWS_TAIL = N - NS * ROWS_PT

E2 = 2 * E
SPW = E2 // NW
KS = 80
NBS = SPW // KS

_MESH = plsc.VectorSubcoreMesh(
    core_axis_name="c", subcore_axis_name="s", num_cores=NC, num_subcores=NS)
_SC_PARAMS = pltpu.CompilerParams(needs_layout_passes=False)


def _zero_acc(zrows, acc, sid):
    rbase = sid * ROWS_PT
    tbase = NS * ROWS_PT
    pltpu.sync_copy(zrows.at[pl.ds(rbase, ROWS_PT)], acc.at[pl.ds(rbase, ROWS_PT)])

    @pl.when(sid == 0)
    def _():
        pltpu.sync_copy(zrows.at[pl.ds(tbase, ROWS_TAIL)],
                        acc.at[pl.ds(tbase, ROWS_TAIL)])


def _copy_out(acc, out, cid, sid):
    rbase = sid * ROWS_PT
    tbase = NS * ROWS_PT
    pltpu.sync_copy(acc.at[pl.ds(rbase, ROWS_PT)],
                    out.at[cid].at[pl.ds(rbase, ROWS_PT)])

    @pl.when(sid == 0)
    def _():
        pltpu.sync_copy(acc.at[pl.ds(tbase, ROWS_TAIL)],
                        out.at[cid].at[pl.ds(tbase, ROWS_TAIL)])


def _agg_body(h, src2, dst3, zrows, agg_out,
              sidx, didx, rows_a, rows_b, acc, sga, sgb, ssa, ssb):
    cid = lax.axis_index("c")
    sid = lax.axis_index("s")
    wid = cid * NS + sid

    _zero_acc(zrows, acc, sid)
    pltpu.sync_copy(src2.at[wid], sidx)
    pltpu.sync_copy(dst3.at[wid], didx)
    plsc.subcore_barrier()

    def gather(j, rows, sem):
        return pltpu.async_copy(h.at[sidx.at[pl.ds(j * KE, KE)]], rows, sem)

    def gather_wait(rows, sem):
        pltpu.make_async_copy(h.at[sidx.at[pl.ds(0, KE)]], rows, sem).wait()

    def scat(j, rows, sem):
        return pltpu.async_copy(rows, acc.at[didx.at[j]], sem, add=True)

    def scat_wait(rows, sem):
        pltpu.make_async_copy(rows, acc.at[didx.at[0]], sem).wait()

    gather(0, rows_a, sga)

    def blk(i, carry):
        a = 2 * i
        b = a + 1
        gather_wait(rows_a, sga)
        gather(b, rows_b, sgb)
        scat(a, rows_a, ssa)
        gather_wait(rows_b, sgb)
        scat_wait(rows_a, ssa)
        gather(jnp.minimum(a + 2, NBE - 1), rows_a, sga)
        scat(b, rows_b, ssb)
        scat_wait(rows_b, ssb)
        return carry

    lax.fori_loop(0, NBE // 2, blk, 0)
    gather_wait(rows_a, sga)
    scat(NBE - 1, rows_a, ssa)
    scat_wait(rows_a, ssa)

    plsc.subcore_barrier()
    _copy_out(acc, agg_out, cid, sid)


_agg = pl.kernel(
    _agg_body,
    out_type=jax.ShapeDtypeStruct((NC, N, D), jnp.float32),
    mesh=_MESH,
    scratch_types=[
        pltpu.VMEM((EPW,), jnp.int32),
        pltpu.VMEM((NBE, KE), jnp.int32),
        pltpu.VMEM((KE, D), jnp.float32),
        pltpu.VMEM((KE, D), jnp.float32),
        pltpu.VMEM_SHARED((N, D), jnp.float32),
        pltpu.SemaphoreType.DMA,
        pltpu.SemaphoreType.DMA,
        pltpu.SemaphoreType.DMA,
        pltpu.SemaphoreType.DMA,
    ],
    compiler_params=_SC_PARAMS,
)


def _deg_body(dst3, zrows, ones_h, deg_out, didx, onesv, dacc, ssa, ssb):
    cid = lax.axis_index("c")
    sid = lax.axis_index("s")
    wid = cid * NS + sid

    _zero_acc(zrows, dacc, sid)
    pltpu.sync_copy(dst3.at[wid], didx)
    pltpu.sync_copy(ones_h, onesv)
    plsc.subcore_barrier()

    def scat(j, sem):
        return pltpu.async_copy(onesv, dacc.at[didx.at[j]], sem, add=True)

    def scat_wait(sem):
        pltpu.make_async_copy(onesv, dacc.at[didx.at[0]], sem).wait()

    def blk(i, carry):
        scat(2 * i, ssa)
        scat(2 * i + 1, ssb)
        scat_wait(ssa)
        scat_wait(ssb)
        return carry

    lax.fori_loop(0, NBE // 2, blk, 0)
    scat(NBE - 1, ssa)
    scat_wait(ssa)

    plsc.subcore_barrier()
    _copy_out(dacc, deg_out, cid, sid)


_deg = pl.kernel(
    _deg_body,
    out_type=jax.ShapeDtypeStruct((NC, N, D), jnp.float32),
    mesh=_MESH,
    scratch_types=[
        pltpu.VMEM((NBE, KE), jnp.int32),
        pltpu.VMEM((KE, D), jnp.float32),
        pltpu.VMEM_SHARED((N, D), jnp.float32),
        pltpu.SemaphoreType.DMA,
        pltpu.SemaphoreType.DMA,
    ],
    compiler_params=_SC_PARAMS,
)


def _score_body(h, src2, dst2, out,
                sidx, didx, sra, dra, srb, drb, sv,
                gsa, gda, gsb, gdb):
    cid = lax.axis_index("c")
    sid = lax.axis_index("s")
    wid = cid * NS + sid

    pltpu.sync_copy(src2.at[wid], sidx)
    pltpu.sync_copy(dst2.at[wid], didx)

    def gathers(j, srows, drows, gs, gd):
        pltpu.async_copy(h.at[sidx.at[pl.ds(j * KS, KS)]], srows, gs)
        pltpu.async_copy(h.at[didx.at[pl.ds(j * KS, KS)]], drows, gd)

    def gathers_wait(srows, drows, gs, gd):
        pltpu.make_async_copy(h.at[sidx.at[pl.ds(0, KS)]], srows, gs).wait()
        pltpu.make_async_copy(h.at[didx.at[pl.ds(0, KS)]], drows, gd).wait()

    lane = lax.iota(jnp.int32, 16)

    def compute(j, srows, drows):
        for g in range(KS // 16):
            score = jnp.zeros((16,), jnp.float32)
            for e in range(16):
                r = g * 16 + e
                p = jnp.zeros((16,), jnp.float32)
                for v in range(D // 16):
                    p = p + (srows[r, pl.ds(v * 16, 16)]
                             * drows[r, pl.ds(v * 16, 16)])
                s = jnp.sum(p)
                score = jnp.where(lane == e, s, score)
            sv[pl.ds(j * KS + g * 16, 16)] = score

    gathers(0, sra, dra, gsa, gda)

    def blk(i, carry):
        a = 2 * i
        b = a + 1
        gathers_wait(sra, dra, gsa, gda)
        gathers(b, srb, drb, gsb, gdb)
        compute(a, sra, dra)
        gathers_wait(srb, drb, gsb, gdb)
        gathers(jnp.minimum(a + 2, NBS - 1), sra, dra, gsa, gda)
        compute(b, srb, drb)
        return carry

    lax.fori_loop(0, NBS // 2, blk, 0)
    gathers_wait(sra, dra, gsa, gda)
    pltpu.sync_copy(sv, out.at[pl.ds(wid * SPW, SPW)])


_score = pl.kernel(
    _score_body,
    out_type=jax.ShapeDtypeStruct((E2,), jnp.float32),
    mesh=_MESH,
    scratch_types=[
        pltpu.VMEM((SPW,), jnp.int32),
        pltpu.VMEM((SPW,), jnp.int32),
        pltpu.VMEM((KS, D), jnp.float32),
        pltpu.VMEM((KS, D), jnp.float32),
        pltpu.VMEM((KS, D), jnp.float32),
        pltpu.VMEM((KS, D), jnp.float32),
        pltpu.VMEM((SPW,), jnp.float32),
        pltpu.SemaphoreType.DMA,
        pltpu.SemaphoreType.DMA,
        pltpu.SemaphoreType.DMA,
        pltpu.SemaphoreType.DMA,
    ],
    compiler_params=_SC_PARAMS,
)


def _combine_body(relu, x, p0, p1, d0, d1, ws, wn, b, out):
    deg = jnp.maximum(d0[:, :1] + d1[:, :1], 1.0)
    hn = (p0[...] + p1[...]) / deg
    r = (jnp.dot(x[...], ws[...], preferred_element_type=jnp.float32,
                 precision=lax.Precision.HIGHEST)
         + jnp.dot(hn, wn[...], preferred_element_type=jnp.float32,
                   precision=lax.Precision.HIGHEST)
         + b[...])
    if relu:
        r = jnp.maximum(r, 0.0)
    out[...] = r


_RB = 1000


def _combine(x, p0, p1, d0, d1, ws, wn, b, relu):
    return pl.pallas_call(
        functools.partial(_combine_body, relu),
        grid=(N // _RB,),
        in_specs=[
            pl.BlockSpec((_RB, D), lambda i: (i, 0)),
            pl.BlockSpec((_RB, D), lambda i: (i, 0)),
            pl.BlockSpec((_RB, D), lambda i: (i, 0)),
            pl.BlockSpec((_RB, D), lambda i: (i, 0)),
            pl.BlockSpec((_RB, D), lambda i: (i, 0)),
            pl.BlockSpec((D, D), lambda i: (0, 0)),
            pl.BlockSpec((D, D), lambda i: (0, 0)),
            pl.BlockSpec((1, D), lambda i: (0, 0)),
        ],
        out_specs=pl.BlockSpec((_RB, D), lambda i: (i, 0)),
        out_shape=jax.ShapeDtypeStruct((N, D), jnp.float32),
    )(x, p0, p1, d0, d1, ws, wn, b)


def kernel(x, edge_index, neg_edge_index, W_self1, W_neigh1, b1,
           W_self2, W_neigh2, b2):
    src = edge_index[0]
    dst = edge_index[1]
    src2 = src.reshape(NW, EPW)
    dst3 = dst.reshape(NW, NBE, KE)
    zrows = jnp.zeros((N, D), jnp.float32)
    ones_h = jnp.ones((KE, D), jnp.float32)

    agg1 = _agg(x, src2, dst3, zrows)
    deg = _deg(dst3, zrows, ones_h)
    h1 = _combine(x, agg1[0], agg1[1], deg[0], deg[1],
                  W_self1, W_neigh1, b1.reshape(1, D), relu=True)
    agg2 = _agg(h1, src2, dst3, zrows)
    h2 = _combine(h1, agg2[0], agg2[1], deg[0], deg[1],
                  W_self2, W_neigh2, b2.reshape(1, D), relu=False)

    srcs2 = jnp.concatenate([src, neg_edge_index[0]]).reshape(NW, SPW)
    dsts2 = jnp.concatenate([dst, neg_edge_index[1]]).reshape(NW, SPW)
    scores = _score(h2, srcs2, dsts2)
    return (scores[:E, None], scores[E:, None])
